# baseline (speedup 1.0000x reference)
import functools

import jax
import jax.numpy as jnp
from jax import lax
from jax.experimental import pallas as pl
from jax.experimental.pallas import tpu as pltpu
from jax.experimental.pallas import tpu_sc as plsc

_NEG = -1e30
_LOG1P = (
    3.910905549409094e-08, 0.9999936302585134, -0.4998254986434647,
    0.33144665224336606, -0.2394333707458602, 0.16499812983396112,
    -0.09229041738050231, 0.03426459995555095, -0.006006605050865348,
)
_LOG1P4 = (
    7.942077648770418e-05, 0.9959657831345109, -0.4650204374456057,
    0.2164487077843725, -0.054370933555584255,
)
_LOGV = (
    -1.5212730017175031, 2.2357796559923986, -0.9022461788064423,
    0.20824503946319362, -0.019632170636695513,
)


def _poly(coefs, x):
    acc = x * jnp.float32(coefs[-1]) + jnp.float32(coefs[-2])
    for c in coefs[-3::-1]:
        acc = acc * x + jnp.float32(c)
    return acc


def _lae(x, y):
    m = jnp.maximum(x, y)
    d = jnp.minimum(x, y) - m
    return m + _poly(_LOG1P, jnp.exp(d))


def kernel(log_probs, targets, input_lengths, target_lengths):
    T, B, C = log_probs.shape
    Lmax = targets.shape[0] // B
    lp_rows = log_probs.reshape(T * B, C)

    info = plsc.get_sparse_core_info()
    NC, L = info.num_cores, info.num_lanes
    RCH = 128
    NLB = Lmax // L
    NBL = (Lmax + L) // L

    mesh = plsc.VectorSubcoreMesh(core_axis_name="c", subcore_axis_name="s")

    @functools.partial(
        pl.kernel, mesh=mesh,
        out_type=jax.ShapeDtypeStruct((B, L), jnp.float32),
        compiler_params=pltpu.CompilerParams(needs_layout_passes=False),
        scratch_types=[
            pltpu.VMEM((T // RCH, RCH), jnp.int32),
            pltpu.VMEM((T, C), jnp.float32),
            pltpu.VMEM((B * Lmax,), jnp.int32),
            pltpu.VMEM((B,), jnp.int32),
            pltpu.VMEM((B,), jnp.int32),
            pltpu.VMEM(((NBL + 1) * L,), jnp.float32),
            pltpu.VMEM((NBL * L,), jnp.float32),
            pltpu.VMEM((NBL * L,), jnp.int32),
            pltpu.VMEM((L,), jnp.float32),
            pltpu.SemaphoreType.DMA,
        ],
    )
    def ctc_sc(lp_hbm, tgt_hbm, il_hbm, tl_hbm, out_hbm,
               rows_v, lp_v, tgt_v, tl_v, il_v, lbuf, bbuf, cbuf, out_v, sem):
        b = lax.axis_index("s") * NC + lax.axis_index("c")
        lane = lax.iota(jnp.int32, L)
        zerov = jnp.zeros((L,), jnp.int32)
        negv = jnp.full((L,), _NEG, jnp.float32)

        per_row = RCH // L
        for k in range(T // L):
            rows_v[k // per_row, pl.ds((k % per_row) * L, L)] = (lane + k * L) * B + b

        cps = [
            pltpu.async_copy(lp_hbm.at[rows_v.at[k]],
                             lp_v.at[pl.ds(k * RCH, RCH)], sem)
            for k in range(T // RCH)
        ]
        pltpu.sync_copy(tgt_hbm, tgt_v)
        pltpu.sync_copy(tl_hbm, tl_v)
        pltpu.sync_copy(il_hbm, il_v)

        bsplat = lax.broadcast(b, (L,))
        tl_b = plsc.load_gather(tl_v, [bsplat])
        il_b = plsc.load_gather(il_v, [bsplat])

        start = jnp.int32(0)
        for k in range(B // L):
            seg = tl_v[pl.ds(k * L, L)]
            start = start + jnp.sum(jnp.where(lane + k * L < b, seg, 0))

        cbuf[pl.ds(0, L)] = jnp.where(lane == 0, jnp.int32(-1), jnp.int32(0))
        for k in range(1, NBL):
            cbuf[pl.ds(k * L, L)] = zerov
        chb = []
        for k in range(NLB):
            j = lane + k * L
            gidx = jnp.clip(start + j, 0, B * Lmax - 1)
            ch = plsc.load_gather(tgt_v, [gidx])
            ch = jnp.where(j < tl_b, ch, 0)
            chb.append(ch)
            plsc.store_scatter(cbuf, [j + 1], ch)
        skipb = []
        for k in range(NLB):
            csh = cbuf[pl.ds(k * L, L)]
            skipb.append((chb[k] != 0) & (chb[k] != csh))

        for k in range(NBL + 1):
            lbuf[pl.ds(k * L, L)] = negv

        cps[0].wait()
        em_b0 = plsc.load_gather(lp_v, [zerov, zerov])
        em_c0 = plsc.load_gather(lp_v, [zerov, chb[0]])
        bl = [jnp.where(lane == 0, em_b0, negv)] + [negv] * (NBL - 1)
        lb = [jnp.where((lane == 0) & (tl_b > 0), em_c0, negv)] + [negv] * (NLB - 1)

        il_s = lax.reduce_max(il_b, axes=(0,))

        def step(t, carry):
            bl = carry[:NBL]
            lb = carry[NBL:]
            for k in range(NLB):
                lbuf[pl.ds(k * L + 1, L)] = lb[k]
            ts = lax.broadcast(t, (L,))
            em_b = plsc.load_gather(lp_v, [ts, zerov])
            lsh = [lbuf[pl.ds(k * L, L)] for k in range(NBL)]
            nbl = []
            for k in range(NBL):
                m = jnp.maximum(bl[k], lsh[k])
                d = jnp.minimum(bl[k], lsh[k]) - m
                nbl.append(m + _poly(_LOG1P4, jnp.exp(d)) + em_b)
            nlb = []
            for k in range(NLB):
                em = plsc.load_gather(lp_v, [ts, chb[k]])
                m1 = jnp.maximum(lb[k], bl[k])
                d1 = jnp.minimum(lb[k], bl[k]) - m1
                lp1 = m1 + _poly(_LOG1P4, jnp.exp(d1))
                s2 = jnp.where(skipb[k], lsh[k], negv)
                m2 = jnp.maximum(lp1, s2)
                d2 = jnp.minimum(lp1, s2) - m2
                nlb.append(m2 + _poly(_LOG1P4, jnp.exp(d2)) + em)
            return (*nbl, *nlb)

        aa = (*bl, *lb)
        for ph in range(T // RCH):
            if ph:
                cps[ph].wait()
            lo = jnp.maximum(jnp.int32(1), jnp.int32(ph * RCH))
            hi = jnp.minimum(il_s, jnp.int32((ph + 1) * RCH))
            aa = lax.fori_loop(lo, hi, step, aa)

        for k in range(NBL):
            bbuf[pl.ds(k * L, L)] = aa[k]
        for k in range(NLB):
            lbuf[pl.ds(k * L + 1, L)] = aa[NBL + k]
        ra = plsc.load_gather(bbuf, [tl_b])
        rb = plsc.load_gather(lbuf, [jnp.maximum(tl_b - 1, jnp.int32(0)) + 1])

        total = jnp.where(tl_b > 0, _lae(ra, rb), ra)
        loss = -total
        bad = (loss != loss) | (jnp.abs(loss) == jnp.float32(jnp.inf))
        out_v[...] = jnp.where(bad, jnp.float32(0.0), loss)
        pltpu.sync_copy(out_v, out_hbm.at[b])

    losses = ctc_sc(lp_rows, targets, input_lengths, target_lengths)
    safe = jnp.maximum(target_lengths, 1).astype(jnp.float32)
    return jnp.mean(losses[:, 0] / safe)


# speedup vs baseline: 1.0512x; 1.0512x over previous
import functools

import jax
import jax.numpy as jnp
from jax import lax
from jax.experimental import pallas as pl
from jax.experimental.pallas import tpu as pltpu
from jax.experimental.pallas import tpu_sc as plsc

_NEG = -1e30
_LOG1P = (
    3.910905549409094e-08, 0.9999936302585134, -0.4998254986434647,
    0.33144665224336606, -0.2394333707458602, 0.16499812983396112,
    -0.09229041738050231, 0.03426459995555095, -0.006006605050865348,
)
_LOG1P4 = (
    7.942077648770418e-05, 0.9959657831345109, -0.4650204374456057,
    0.2164487077843725, -0.054370933555584255,
)
_LOGV = (
    -1.5212730017175031, 2.2357796559923986, -0.9022461788064423,
    0.20824503946319362, -0.019632170636695513,
)


def _poly(coefs, x):
    acc = x * jnp.float32(coefs[-1]) + jnp.float32(coefs[-2])
    for c in coefs[-3::-1]:
        acc = acc * x + jnp.float32(c)
    return acc


def _lae(x, y):
    m = jnp.maximum(x, y)
    d = jnp.minimum(x, y) - m
    return m + _poly(_LOG1P, jnp.exp(d))


def kernel(log_probs, targets, input_lengths, target_lengths):
    T, B, C = log_probs.shape
    Lmax = targets.shape[0] // B
    lp_rows = log_probs.reshape(T * B, C)

    info = plsc.get_sparse_core_info()
    NC, L = info.num_cores, info.num_lanes
    RCH = 128
    NLB = Lmax // L
    NBL = (Lmax + L) // L

    mesh = plsc.VectorSubcoreMesh(core_axis_name="c", subcore_axis_name="s")

    @functools.partial(
        pl.kernel, mesh=mesh,
        out_type=jax.ShapeDtypeStruct((B, L), jnp.float32),
        compiler_params=pltpu.CompilerParams(needs_layout_passes=False),
        scratch_types=[
            pltpu.VMEM((T // RCH, RCH), jnp.int32),
            pltpu.VMEM((T, C), jnp.float32),
            pltpu.VMEM((B * Lmax,), jnp.int32),
            pltpu.VMEM((B,), jnp.int32),
            pltpu.VMEM((B,), jnp.int32),
            pltpu.VMEM(((NBL + 1) * L,), jnp.float32),
            pltpu.VMEM((NBL * L,), jnp.float32),
            pltpu.VMEM((NBL * L,), jnp.int32),
            pltpu.VMEM((L,), jnp.float32),
            pltpu.SemaphoreType.DMA,
        ],
    )
    def ctc_sc(lp_hbm, tgt_hbm, il_hbm, tl_hbm, out_hbm,
               rows_v, lp_v, tgt_v, tl_v, il_v, lbuf, bbuf, cbuf, out_v, sem):
        b = lax.axis_index("s") * NC + lax.axis_index("c")
        lane = lax.iota(jnp.int32, L)
        zerov = jnp.zeros((L,), jnp.int32)
        negv = jnp.full((L,), _NEG, jnp.float32)

        per_row = RCH // L
        for k in range(T // L):
            rows_v[k // per_row, pl.ds((k % per_row) * L, L)] = (lane + k * L) * B + b

        cps = [
            pltpu.async_copy(lp_hbm.at[rows_v.at[k]],
                             lp_v.at[pl.ds(k * RCH, RCH)], sem)
            for k in range(T // RCH)
        ]
        pltpu.sync_copy(tgt_hbm, tgt_v)
        pltpu.sync_copy(tl_hbm, tl_v)
        pltpu.sync_copy(il_hbm, il_v)

        bsplat = lax.broadcast(b, (L,))
        tl_b = plsc.load_gather(tl_v, [bsplat])
        il_b = plsc.load_gather(il_v, [bsplat])

        start = jnp.int32(0)
        for k in range(B // L):
            seg = tl_v[pl.ds(k * L, L)]
            start = start + jnp.sum(jnp.where(lane + k * L < b, seg, 0))

        cbuf[pl.ds(0, L)] = jnp.where(lane == 0, jnp.int32(-1), jnp.int32(0))
        for k in range(1, NBL):
            cbuf[pl.ds(k * L, L)] = zerov
        chb = []
        for k in range(NLB):
            j = lane + k * L
            gidx = jnp.clip(start + j, 0, B * Lmax - 1)
            ch = plsc.load_gather(tgt_v, [gidx])
            ch = jnp.where(j < tl_b, ch, 0)
            chb.append(ch)
            plsc.store_scatter(cbuf, [j + 1], ch)
        skipb = []
        for k in range(NLB):
            csh = cbuf[pl.ds(k * L, L)]
            skipb.append((chb[k] != 0) & (chb[k] != csh))

        for k in range(NBL + 1):
            lbuf[pl.ds(k * L, L)] = negv

        cps[0].wait()
        em_b0 = plsc.load_gather(lp_v, [zerov, zerov])
        em_c0 = plsc.load_gather(lp_v, [zerov, chb[0]])
        bl = [jnp.where(lane == 0, em_b0, negv)] + [negv] * (NBL - 1)
        lb = [jnp.where((lane == 0) & (tl_b > 0), em_c0, negv)] + [negv] * (NLB - 1)

        il_s = lax.reduce_max(il_b, axes=(0,))

        def step(t, carry):
            bl = carry[:NBL]
            lb = carry[NBL:]
            for k in range(NLB):
                lbuf[pl.ds(k * L + 1, L)] = lb[k]
            ts = lax.broadcast(t, (L,))
            em_b = plsc.load_gather(lp_v, [ts, zerov])
            lsh = [lbuf[pl.ds(k * L, L)] for k in range(NBL)]
            nbl = []
            for k in range(NBL):
                m = jnp.maximum(bl[k], lsh[k])
                d = jnp.minimum(bl[k], lsh[k]) - m
                nbl.append(m + _poly(_LOG1P4, jnp.exp(d)) + em_b)
            nlb = []
            for k in range(NLB):
                em = plsc.load_gather(lp_v, [ts, chb[k]])
                s2 = jnp.where(skipb[k], lsh[k], negv)
                m = jnp.maximum(jnp.maximum(lb[k], bl[k]), s2)
                v = jnp.exp(lb[k] - m) + jnp.exp(bl[k] - m) + jnp.exp(s2 - m)
                nlb.append(m + _poly(_LOGV, v) + em)
            return (*nbl, *nlb)

        aa = (*bl, *lb)
        for ph in range(T // RCH):
            if ph:
                cps[ph].wait()
            lo = jnp.maximum(jnp.int32(1), jnp.int32(ph * RCH))
            hi = jnp.minimum(il_s, jnp.int32((ph + 1) * RCH))
            aa = lax.fori_loop(lo, hi, step, aa)

        for k in range(NBL):
            bbuf[pl.ds(k * L, L)] = aa[k]
        for k in range(NLB):
            lbuf[pl.ds(k * L + 1, L)] = aa[NBL + k]
        ra = plsc.load_gather(bbuf, [tl_b])
        rb = plsc.load_gather(lbuf, [jnp.maximum(tl_b - 1, jnp.int32(0)) + 1])

        total = jnp.where(tl_b > 0, _lae(ra, rb), ra)
        loss = -total
        bad = (loss != loss) | (jnp.abs(loss) == jnp.float32(jnp.inf))
        out_v[...] = jnp.where(bad, jnp.float32(0.0), loss)
        pltpu.sync_copy(out_v, out_hbm.at[b])

    losses = ctc_sc(lp_rows, targets, input_lengths, target_lengths)
    safe = jnp.maximum(target_lengths, 1).astype(jnp.float32)
    return jnp.mean(losses[:, 0] / safe)


# speedup vs baseline: 1.0747x; 1.0224x over previous
import functools

import jax
import jax.numpy as jnp
from jax import lax
from jax.experimental import pallas as pl
from jax.experimental.pallas import tpu as pltpu
from jax.experimental.pallas import tpu_sc as plsc

_NEG = -1e30
_LOG1P = (
    3.910905549409094e-08, 0.9999936302585134, -0.4998254986434647,
    0.33144665224336606, -0.2394333707458602, 0.16499812983396112,
    -0.09229041738050231, 0.03426459995555095, -0.006006605050865348,
)
_LOG1P4 = (
    7.942077648770418e-05, 0.9959657831345109, -0.4650204374456057,
    0.2164487077843725, -0.054370933555584255,
)
_LOGV = (
    -1.5212730017175031, 2.2357796559923986, -0.9022461788064423,
    0.20824503946319362, -0.019632170636695513,
)


def _poly(coefs, x):
    acc = x * jnp.float32(coefs[-1]) + jnp.float32(coefs[-2])
    for c in coefs[-3::-1]:
        acc = acc * x + jnp.float32(c)
    return acc


def _lae(x, y):
    m = jnp.maximum(x, y)
    d = jnp.minimum(x, y) - m
    return m + _poly(_LOG1P, jnp.exp(d))


def kernel(log_probs, targets, input_lengths, target_lengths):
    T, B, C = log_probs.shape
    Lmax = targets.shape[0] // B
    lp_rows = log_probs.reshape(T * B, C)

    info = plsc.get_sparse_core_info()
    NC, L = info.num_cores, info.num_lanes
    RCH = 128
    NLB = Lmax // L
    NBL = (Lmax + L) // L

    mesh = plsc.VectorSubcoreMesh(core_axis_name="c", subcore_axis_name="s")

    @functools.partial(
        pl.kernel, mesh=mesh,
        out_type=jax.ShapeDtypeStruct((B, L), jnp.float32),
        compiler_params=pltpu.CompilerParams(needs_layout_passes=False),
        scratch_types=[
            pltpu.VMEM((T // RCH, RCH), jnp.int32),
            pltpu.VMEM((T, C), jnp.float32),
            pltpu.VMEM((B * Lmax,), jnp.int32),
            pltpu.VMEM((B,), jnp.int32),
            pltpu.VMEM((B,), jnp.int32),
            pltpu.VMEM(((NBL + 1) * L,), jnp.float32),
            pltpu.VMEM((NBL * L,), jnp.float32),
            pltpu.VMEM((NBL * L,), jnp.int32),
            pltpu.VMEM((L,), jnp.float32),
            pltpu.SemaphoreType.DMA,
        ],
    )
    def ctc_sc(lp_hbm, tgt_hbm, il_hbm, tl_hbm, out_hbm,
               rows_v, lp_v, tgt_v, tl_v, il_v, lbuf, bbuf, cbuf, out_v, sem):
        b = lax.axis_index("s") * NC + lax.axis_index("c")
        lane = lax.iota(jnp.int32, L)
        zerov = jnp.zeros((L,), jnp.int32)
        negv = jnp.full((L,), _NEG, jnp.float32)

        per_row = RCH // L
        for k in range(T // L):
            rows_v[k // per_row, pl.ds((k % per_row) * L, L)] = (lane + k * L) * B + b

        cps = [
            pltpu.async_copy(lp_hbm.at[rows_v.at[k]],
                             lp_v.at[pl.ds(k * RCH, RCH)], sem)
            for k in range(T // RCH)
        ]
        pltpu.sync_copy(tgt_hbm, tgt_v)
        pltpu.sync_copy(tl_hbm, tl_v)
        pltpu.sync_copy(il_hbm, il_v)

        bsplat = lax.broadcast(b, (L,))
        tl_b = plsc.load_gather(tl_v, [bsplat])
        il_b = plsc.load_gather(il_v, [bsplat])

        start = jnp.int32(0)
        for k in range(B // L):
            seg = tl_v[pl.ds(k * L, L)]
            start = start + jnp.sum(jnp.where(lane + k * L < b, seg, 0))

        cbuf[pl.ds(0, L)] = jnp.where(lane == 0, jnp.int32(-1), jnp.int32(0))
        for k in range(1, NBL):
            cbuf[pl.ds(k * L, L)] = zerov
        chb = []
        for k in range(NLB):
            j = lane + k * L
            gidx = jnp.clip(start + j, 0, B * Lmax - 1)
            ch = plsc.load_gather(tgt_v, [gidx])
            ch = jnp.where(j < tl_b, ch, 0)
            chb.append(ch)
            plsc.store_scatter(cbuf, [j + 1], ch)
        skipb = []
        for k in range(NLB):
            csh = cbuf[pl.ds(k * L, L)]
            skipb.append((chb[k] != 0) & (chb[k] != csh))

        for k in range(NBL + 1):
            lbuf[pl.ds(k * L, L)] = negv

        cps[0].wait()
        em_b0 = plsc.load_gather(lp_v, [zerov, zerov])
        em_c0 = plsc.load_gather(lp_v, [zerov, chb[0]])
        bl = [jnp.where(lane == 0, em_b0, negv)] + [negv] * (NBL - 1)
        lb = [jnp.where((lane == 0) & (tl_b > 0), em_c0, negv)] + [negv] * (NLB - 1)

        il_s = lax.reduce_max(il_b, axes=(0,))

        def step(t, carry):
            bl = carry[:NBL]
            lb = carry[NBL:]
            for k in range(NLB):
                lbuf[pl.ds(k * L + 1, L)] = lb[k]
            ts = lax.broadcast(t, (L,))
            em_b = plsc.load_gather(lp_v, [ts, zerov])
            lsh = [lbuf[pl.ds(k * L, L)] for k in range(NBL)]
            nbl = []
            for k in range(NBL):
                m = jnp.maximum(bl[k], lsh[k])
                d = jnp.minimum(bl[k], lsh[k]) - m
                nbl.append(m + _poly(_LOG1P4, jnp.exp(d)) + em_b)
            nlb = []
            for k in range(NLB):
                em = plsc.load_gather(lp_v, [ts, chb[k]])
                s2 = jnp.where(skipb[k], lsh[k], negv)
                m = jnp.maximum(jnp.maximum(lb[k], bl[k]), s2)
                v = jnp.exp(lb[k] - m) + jnp.exp(bl[k] - m) + jnp.exp(s2 - m)
                nlb.append(m + _poly(_LOGV, v) + em)
            return (*nbl, *nlb)

        for cp in cps[1:]:
            cp.wait()
        aa = lax.fori_loop(1, il_s, step, (*bl, *lb))

        for k in range(NBL):
            bbuf[pl.ds(k * L, L)] = aa[k]
        for k in range(NLB):
            lbuf[pl.ds(k * L + 1, L)] = aa[NBL + k]
        ra = plsc.load_gather(bbuf, [tl_b])
        rb = plsc.load_gather(lbuf, [jnp.maximum(tl_b - 1, jnp.int32(0)) + 1])

        total = jnp.where(tl_b > 0, _lae(ra, rb), ra)
        loss = -total
        bad = (loss != loss) | (jnp.abs(loss) == jnp.float32(jnp.inf))
        out_v[...] = jnp.where(bad, jnp.float32(0.0), loss)
        pltpu.sync_copy(out_v, out_hbm.at[b])

    losses = ctc_sc(lp_rows, targets, input_lengths, target_lengths)
    safe = jnp.maximum(target_lengths, 1).astype(jnp.float32)
    return jnp.mean(losses[:, 0] / safe)
